# baseline (speedup 1.0000x reference)
import functools

import jax
import jax.numpy as jnp
from jax import lax
from jax.experimental import pallas as pl
from jax.experimental.pallas import tpu as pltpu
from jax.experimental.pallas import tpu_sc as plsc

NC = 2
NS = 16
NW = NC * NS
L = 16

_MESH = plsc.VectorSubcoreMesh(
    core_axis_name="c", subcore_axis_name="s", num_cores=NC, num_subcores=NS
)
_SC_PARAMS = pltpu.CompilerParams(
    needs_layout_passes=False, use_tc_tiling_on_sc=False
)


def _degree_kernel(e16, np_pad):
    @functools.partial(
        pl.kernel,
        mesh=_MESH,
        out_type=jax.ShapeDtypeStruct((2, NW, np_pad), jnp.float32),
        scratch_types=[
            pltpu.VMEM((e16, L), jnp.int32),
            pltpu.VMEM((e16, L), jnp.int32),
            pltpu.VMEM((np_pad,), jnp.float32),
            pltpu.VMEM((np_pad,), jnp.float32),
        ],
        compiler_params=_SC_PARAMS,
    )
    def deg_kernel(src_hbm, dst_hbm, out_hbm, sidx, didx, hs, hd):
        c = lax.axis_index("c")
        s = lax.axis_index("s")
        wid = s * NC + c
        pltpu.sync_copy(src_hbm.at[wid], sidx)
        pltpu.sync_copy(dst_hbm.at[wid], didx)
        zeros = jnp.zeros((L,), jnp.float32)

        def zbody(k, carry):
            hs[pl.ds(k * L, L)] = zeros
            hd[pl.ds(k * L, L)] = zeros
            return carry

        lax.fori_loop(0, np_pad // L, zbody, 0)
        ones = jnp.ones((L,), jnp.float32)

        def ebody(j, carry):
            plsc.addupdate_scatter(hs, [sidx[j]], ones)
            plsc.addupdate_scatter(hd, [didx[j]], ones)
            return carry

        lax.fori_loop(0, e16, ebody, 0)
        pltpu.sync_copy(hs, out_hbm.at[0, wid])
        pltpu.sync_copy(hd, out_hbm.at[1, wid])

    return deg_kernel


def _agg_kernel(nch, ch, np_pad, dh):
    rpt = np_pad // NS

    @functools.partial(
        pl.kernel,
        mesh=_MESH,
        out_type=jax.ShapeDtypeStruct((NC, np_pad, dh), jnp.float32),
        scratch_types=[
            pltpu.VMEM((nch // 2, ch), jnp.int32),
            pltpu.VMEM((nch // 2, ch), jnp.int32),
            pltpu.VMEM((ch, dh), jnp.float32),
            pltpu.VMEM((ch, dh), jnp.float32),
            pltpu.VMEM_SHARED((np_pad, dh), jnp.float32),
            pltpu.VMEM_SHARED((np_pad, dh), jnp.float32),
            pltpu.SemaphoreType.DMA,
            pltpu.SemaphoreType.DMA,
        ],
        compiler_params=_SC_PARAMS,
    )
    def agg_kernel(
        h_hbm, src_hbm, dst_hbm, zmat_hbm, out_hbm,
        sidx, didx, rows0, rows1, sh, acc, sem0, sem1,
    ):
        c = lax.axis_index("c")
        s = lax.axis_index("s")
        pltpu.sync_copy(h_hbm.at[c].at[pl.ds(s * rpt, rpt)], sh.at[pl.ds(s * rpt, rpt)])
        pltpu.sync_copy(zmat_hbm.at[pl.ds(s * rpt, rpt)], acc.at[pl.ds(s * rpt, rpt)])
        plsc.subcore_barrier()

        nh = nch // 2
        for hh in range(2):
            pltpu.sync_copy(src_hbm.at[s, hh], sidx)
            pltpu.sync_copy(dst_hbm.at[s, hh], didx)

            pltpu.async_copy(sh.at[sidx.at[0]], rows0, sem0)
            pltpu.async_copy(sh.at[sidx.at[1]], rows1, sem1)

            def ebody(k, carry):
                j = 2 * k
                pltpu.make_async_copy(sh.at[sidx.at[j]], rows0, sem0).wait()
                pltpu.sync_copy(rows0, acc.at[didx.at[j]], add=True)

                @pl.when(k < nh // 2 - 1)
                def _():
                    pltpu.async_copy(sh.at[sidx.at[j + 2]], rows0, sem0)

                pltpu.make_async_copy(sh.at[sidx.at[j + 1]], rows1, sem1).wait()
                pltpu.sync_copy(rows1, acc.at[didx.at[j + 1]], add=True)

                @pl.when(k < nh // 2 - 1)
                def _():
                    pltpu.async_copy(sh.at[sidx.at[j + 3]], rows1, sem1)

                return carry

            lax.fori_loop(0, nh // 2, ebody, 0)
        plsc.subcore_barrier()
        pltpu.sync_copy(
            acc.at[pl.ds(s * rpt, rpt)], out_hbm.at[c].at[pl.ds(s * rpt, rpt)]
        )

    return agg_kernel


def _scale_body(hist_ref, x_ref, o_ref):
    deg = jnp.sum(hist_ref[0], axis=0)
    norm = lax.rsqrt(jnp.maximum(deg, 1.0))
    h = x_ref[...] * norm[:, None]
    dh = h.shape[-1] // 2
    o_ref[0] = h[:, :dh]
    o_ref[1] = h[:, dh:]


def _final_body(parts_ref, hist_ref, w_ref, b_ref, o_ref):
    p = parts_ref[...]
    agg = jnp.concatenate([p[0], p[1]], axis=1)
    deg = jnp.sum(hist_ref[1], axis=0)
    norm = lax.rsqrt(jnp.maximum(deg, 1.0))
    z = jnp.dot(agg * norm[:, None], w_ref[...], preferred_element_type=jnp.float32)
    o_ref[...] = jnp.maximum(z + b_ref[...], 0.0)


def kernel(x, edge_index, W, b):
    n, d = x.shape
    dh = d // 2
    e = edge_index.shape[1]
    np_pad = ((n + 1023) // 1024) * 1024
    ept = e // NS
    ch = 100
    nch = ept // ch

    src = edge_index[0]
    dst = edge_index[1]
    src16 = src.reshape(NW, e // NW // L, L)
    dst16 = dst.reshape(NW, e // NW // L, L)

    hist = _degree_kernel(e // NW // L, np_pad)(src16, dst16)

    x_pad = jnp.pad(x, ((0, np_pad - n), (0, 0)))
    blk = 1024
    h2 = pl.pallas_call(
        _scale_body,
        grid=(np_pad // blk,),
        in_specs=[
            pl.BlockSpec((2, NW, blk), lambda i: (0, 0, i)),
            pl.BlockSpec((blk, d), lambda i: (i, 0)),
        ],
        out_specs=pl.BlockSpec((2, blk, dh), lambda i: (0, i, 0)),
        out_shape=jax.ShapeDtypeStruct((2, np_pad, dh), jnp.float32),
    )(hist, x_pad)

    zmat = jnp.zeros((np_pad, dh), jnp.float32)
    parts = _agg_kernel(nch, ch, np_pad, dh)(
        h2, src.reshape(NS, 2, nch // 2, ch), dst.reshape(NS, 2, nch // 2, ch), zmat
    )

    out = pl.pallas_call(
        _final_body,
        grid=(np_pad // blk,),
        in_specs=[
            pl.BlockSpec((NC, blk, dh), lambda i: (0, i, 0)),
            pl.BlockSpec((2, NW, blk), lambda i: (0, 0, i)),
            pl.BlockSpec((d, d), lambda i: (0, 0)),
            pl.BlockSpec((1, d), lambda i: (0, 0)),
        ],
        out_specs=pl.BlockSpec((blk, d), lambda i: (i, 0)),
        out_shape=jax.ShapeDtypeStruct((np_pad, d), jnp.float32),
    )(parts, hist, W, b.reshape(1, d))
    return out[:n]


# speedup vs baseline: 1.3365x; 1.3365x over previous
import functools

import jax
import jax.numpy as jnp
from jax import lax
from jax.experimental import pallas as pl
from jax.experimental.pallas import tpu as pltpu
from jax.experimental.pallas import tpu_sc as plsc

NC = 2
NS = 16
NW = NC * NS
L = 16

_MESH = plsc.VectorSubcoreMesh(
    core_axis_name="c", subcore_axis_name="s", num_cores=NC, num_subcores=NS
)
_SC_PARAMS = pltpu.CompilerParams(
    needs_layout_passes=False, use_tc_tiling_on_sc=False
)


def _degree_kernel(e16, np_pad):
    @functools.partial(
        pl.kernel,
        mesh=_MESH,
        out_type=jax.ShapeDtypeStruct((2, NW, np_pad), jnp.float32),
        scratch_types=[
            pltpu.VMEM((e16, L), jnp.int32),
            pltpu.VMEM((e16, L), jnp.int32),
            pltpu.VMEM((np_pad,), jnp.float32),
            pltpu.VMEM((np_pad,), jnp.float32),
            pltpu.SemaphoreType.DMA,
        ],
        compiler_params=_SC_PARAMS,
    )
    def deg_kernel(src_hbm, dst_hbm, zrow_hbm, out_hbm, sidx, didx, hs, hd, sem):
        c = lax.axis_index("c")
        s = lax.axis_index("s")
        wid = s * NC + c
        pltpu.async_copy(src_hbm.at[wid], sidx, sem)
        pltpu.async_copy(dst_hbm.at[wid], didx, sem)
        pltpu.sync_copy(zrow_hbm.at[0], hs)
        pltpu.sync_copy(zrow_hbm.at[1], hd)
        pltpu.make_async_copy(src_hbm.at[wid], sidx, sem).wait()
        pltpu.make_async_copy(dst_hbm.at[wid], didx, sem).wait()
        ones = jnp.ones((L,), jnp.float32)

        @plsc.parallel_loop(0, e16, unroll=4)
        def _(j):
            plsc.addupdate_scatter(hs, [sidx[j]], ones)
            plsc.addupdate_scatter(hd, [didx[j]], ones)

        pltpu.sync_copy(hs, out_hbm.at[0, wid])
        pltpu.sync_copy(hd, out_hbm.at[1, wid])

    return deg_kernel


def _agg_kernel(nch, ch, np_pad, d):
    rpt = np_pad // NS

    @functools.partial(
        pl.kernel,
        mesh=_MESH,
        out_type=jax.ShapeDtypeStruct((NC, np_pad, d), jnp.float32),
        scratch_types=[
            pltpu.VMEM((nch, ch), jnp.int32),
            pltpu.VMEM((nch, ch), jnp.int32),
            pltpu.VMEM((ch, d), jnp.float32),
            pltpu.VMEM((ch, d), jnp.float32),
            pltpu.VMEM_SHARED((np_pad, d), jnp.float32),
            pltpu.SemaphoreType.DMA,
            pltpu.SemaphoreType.DMA,
        ],
        compiler_params=_SC_PARAMS,
    )
    def agg_kernel(h_hbm, src_hbm, dst_hbm, zmat_hbm, out_hbm,
                   sidx, didx, rows0, rows1, acc, sem0, sem1):
        c = lax.axis_index("c")
        s = lax.axis_index("s")
        wid = s * NC + c
        pltpu.sync_copy(src_hbm.at[wid], sidx)
        pltpu.sync_copy(dst_hbm.at[wid], didx)
        pltpu.sync_copy(zmat_hbm.at[pl.ds(s * rpt, rpt)], acc.at[pl.ds(s * rpt, rpt)])
        plsc.subcore_barrier()

        pltpu.async_copy(h_hbm.at[sidx.at[0]], rows0, sem0)
        pltpu.async_copy(h_hbm.at[sidx.at[1]], rows1, sem1)

        def ebody(k, carry):
            j = 2 * k
            pltpu.make_async_copy(h_hbm.at[sidx.at[j]], rows0, sem0).wait()
            pltpu.sync_copy(rows0, acc.at[didx.at[j]], add=True)

            @pl.when(k < nch // 2 - 1)
            def _():
                pltpu.async_copy(h_hbm.at[sidx.at[j + 2]], rows0, sem0)

            pltpu.make_async_copy(h_hbm.at[sidx.at[j + 1]], rows1, sem1).wait()
            pltpu.sync_copy(rows1, acc.at[didx.at[j + 1]], add=True)

            @pl.when(k < nch // 2 - 1)
            def _():
                pltpu.async_copy(h_hbm.at[sidx.at[j + 3]], rows1, sem1)

            return carry

        lax.fori_loop(0, nch // 2, ebody, 0)
        plsc.subcore_barrier()
        pltpu.sync_copy(
            acc.at[pl.ds(s * rpt, rpt)], out_hbm.at[c].at[pl.ds(s * rpt, rpt)]
        )

    return agg_kernel


def _scale_body(hist_ref, x_ref, o_ref):
    deg = jnp.sum(hist_ref[0], axis=0)
    norm = lax.rsqrt(jnp.maximum(deg, 1.0))
    o_ref[...] = x_ref[...] * norm[:, None]


def _final_body(parts_ref, hist_ref, w_ref, b_ref, o_ref):
    p = parts_ref[...]
    agg = p[0] + p[1]
    deg = jnp.sum(hist_ref[1], axis=0)
    norm = lax.rsqrt(jnp.maximum(deg, 1.0))
    z = jnp.dot(agg * norm[:, None], w_ref[...], preferred_element_type=jnp.float32)
    o_ref[...] = jnp.maximum(z + b_ref[...], 0.0)


def kernel(x, edge_index, W, b):
    n, d = x.shape
    e = edge_index.shape[1]
    np_pad = ((n + 1023) // 1024) * 1024
    ept = e // NW
    ch = 100
    nch = ept // ch

    src = edge_index[0]
    dst = edge_index[1]
    src16 = src.reshape(NW, ept // L, L)
    dst16 = dst.reshape(NW, ept // L, L)

    zrow = jnp.zeros((2, np_pad), jnp.float32)
    hist = _degree_kernel(ept // L, np_pad)(src16, dst16, zrow)

    blk = 1024
    h = pl.pallas_call(
        _scale_body,
        grid=(np_pad // blk,),
        in_specs=[
            pl.BlockSpec((2, NW, blk), lambda i: (0, 0, i)),
            pl.BlockSpec((blk, d), lambda i: (i, 0)),
        ],
        out_specs=pl.BlockSpec((blk, d), lambda i: (i, 0)),
        out_shape=jax.ShapeDtypeStruct((np_pad, d), jnp.float32),
    )(hist, x)

    zmat = jnp.zeros((np_pad, d), jnp.float32)
    parts = _agg_kernel(nch, ch, np_pad, d)(
        h, src.reshape(NW, nch, ch), dst.reshape(NW, nch, ch), zmat
    )

    out = pl.pallas_call(
        _final_body,
        grid=(np_pad // blk,),
        in_specs=[
            pl.BlockSpec((NC, blk, d), lambda i: (0, i, 0)),
            pl.BlockSpec((2, NW, blk), lambda i: (0, 0, i)),
            pl.BlockSpec((d, d), lambda i: (0, 0)),
            pl.BlockSpec((1, d), lambda i: (0, 0)),
        ],
        out_specs=pl.BlockSpec((blk, d), lambda i: (i, 0)),
        out_shape=jax.ShapeDtypeStruct((n, d), jnp.float32),
    )(parts, hist, W, b.reshape(1, d))
    return out
